# baseline (speedup 1.0000x reference)
import functools

import jax
import jax.numpy as jnp
from jax import lax
from jax.experimental import pallas as pl
from jax.experimental.pallas import tpu as pltpu
from jax.experimental.pallas import tpu_sc as plsc

MAXV = 147
KPAD = 448
DBLK = 16


def _sc_body(tab_hbm, idx_hbm, out_hbm, tab_v, idx_v, row_v, *,
             n_workers, b_per_w, n_chan, pixels, d_dim):
    wid = lax.axis_index("s") * plsc.get_sparse_core_info().num_cores \
        + lax.axis_index("c")
    pltpu.sync_copy(tab_hbm, tab_v)

    def per_image(bi, _):
        b = wid * b_per_w + bi
        pltpu.sync_copy(idx_hbm.at[b], idx_v)

        def per_dblk(db, _):
            def per_d(dl, _):
                d = db * DBLK + dl

                def per_chunk(k, _):
                    acc = jnp.zeros((16,), jnp.float32)
                    base = (d * KPAD).astype(jnp.int32)
                    for c in range(n_chan):
                        iv = idx_v[c, pl.ds(k * 16, 16)]
                        acc = acc + plsc.load_gather(tab_v, [base + iv])
                    row_v[dl, pl.ds(k * 16, 16)] = acc
                    return 0

                lax.fori_loop(0, pixels // 16, per_chunk, 0)
                return 0

            lax.fori_loop(0, DBLK, per_d, 0)
            pltpu.sync_copy(row_v, out_hbm.at[b, pl.ds(db * DBLK, DBLK)])
            return 0

        lax.fori_loop(0, d_dim // DBLK, per_dblk, 0)
        return 0

    lax.fori_loop(0, b_per_w, per_image, 0)


@jax.jit
def kernel(inputs, table):
    B, C, H, W = inputs.shape
    V, D = table.shape
    P = H * W
    maxv = V // C

    idx = inputs.astype(jnp.int32).reshape(B, C, P)

    tabT = jnp.pad(table.T, ((0, 0), (0, KPAD - V))).reshape(-1)
    offs = (jnp.arange(C, dtype=jnp.int32) * maxv)[None, :, None]
    idx = idx + offs

    info = plsc.get_sparse_core_info()
    nw = info.num_cores * info.num_subcores
    b_per_w = B // nw

    sc = functools.partial(
        pl.kernel,
        mesh=plsc.VectorSubcoreMesh(core_axis_name="c", subcore_axis_name="s"),
        out_type=jax.ShapeDtypeStruct((B, D, P), jnp.float32),
        scratch_types=[
            pltpu.VMEM((D * KPAD,), jnp.float32),
            pltpu.VMEM((C, P), jnp.int32),
            pltpu.VMEM((DBLK, P), jnp.float32),
        ],
        compiler_params=pltpu.CompilerParams(needs_layout_passes=False),
    )
    body = functools.partial(
        _sc_body, n_workers=nw, b_per_w=b_per_w, n_chan=C, pixels=P,
        d_dim=D)
    out = sc(body)(tabT, idx)
    return out.reshape(B, D, H, W)


# speedup vs baseline: 3.5295x; 3.5295x over previous
import functools

import jax
import jax.numpy as jnp
from jax import lax
from jax.experimental import pallas as pl
from jax.experimental.pallas import tpu as pltpu
from jax.experimental.pallas import tpu_sc as plsc

MAXV = 147
KPAD = 160
SCKP = 448
DBLK = 16
B_SC = 128


def _tc_body(idx_ref, tab_ref, out_ref, *, t_imgs, n_chan, kpad, pixels):
    iota = jax.lax.broadcasted_iota(jnp.int32, (kpad, pixels), 0)
    iota_bf = iota.astype(jnp.bfloat16)
    one = jnp.bfloat16(1.0)
    zero = jnp.bfloat16(0.0)
    for t in range(t_imgs):
        hots = []
        for c in range(n_chan):
            idx_bf = idx_ref[t, c, :].astype(jnp.bfloat16)
            d = iota_bf - idx_bf[None, :]
            hots.append(jnp.maximum(one - jnp.abs(d), zero))
        onehot = jnp.concatenate(hots, axis=0)
        out_ref[t] = jnp.dot(tab_ref[...], onehot,
                             preferred_element_type=jnp.float32)


def _sc_body(tab_hbm, idx_hbm, out_hbm, tab_v, idx_v, row_v, *,
             b_per_w, n_chan, pixels, d_dim):
    wid = lax.axis_index("s") * plsc.get_sparse_core_info().num_cores \
        + lax.axis_index("c")
    pltpu.sync_copy(tab_hbm, tab_v)

    def per_image(bi, _):
        b = wid * b_per_w + bi
        pltpu.sync_copy(idx_hbm.at[b], idx_v)

        def per_dblk(db, _):
            def per_d(dl, _):
                d = db * DBLK + dl

                def per_chunk(k, _):
                    acc = jnp.zeros((16,), jnp.float32)
                    base = (d * SCKP).astype(jnp.int32)
                    for c in range(n_chan):
                        iv = idx_v[c, pl.ds(k * 16, 16)]
                        acc = acc + plsc.load_gather(tab_v, [base + iv])
                    row_v[dl, pl.ds(k * 16, 16)] = acc
                    return 0

                lax.fori_loop(0, pixels // 16, per_chunk, 0)
                return 0

            lax.fori_loop(0, DBLK, per_d, 0)
            pltpu.sync_copy(row_v, out_hbm.at[b, pl.ds(db * DBLK, DBLK)])
            return 0

        lax.fori_loop(0, d_dim // DBLK, per_dblk, 0)
        return 0

    lax.fori_loop(0, b_per_w, per_image, 0)


@jax.jit
def kernel(inputs, table):
    B, C, H, W = inputs.shape
    V, D = table.shape
    P = H * W
    maxv = V // C

    idx = inputs.astype(jnp.int32).reshape(B, C, P)
    b_tc = B - B_SC

    tab = table.reshape(C, maxv, D)
    tab = jnp.pad(tab, ((0, 0), (0, KPAD - maxv), (0, 0)))
    tabT = jnp.transpose(tab, (2, 0, 1)).reshape(D, C * KPAD)
    tabT = tabT.astype(jnp.bfloat16)

    T = 32
    out_tc = pl.pallas_call(
        functools.partial(_tc_body, t_imgs=T, n_chan=C, kpad=KPAD, pixels=P),
        grid=(b_tc // T,),
        in_specs=[
            pl.BlockSpec((T, C, P), lambda i: (i, 0, 0)),
            pl.BlockSpec((D, C * KPAD), lambda i: (0, 0)),
        ],
        out_specs=pl.BlockSpec((T, D, P), lambda i: (i, 0, 0)),
        out_shape=jax.ShapeDtypeStruct((b_tc, D, P), jnp.float32),
        compiler_params=pltpu.CompilerParams(
            dimension_semantics=("parallel",)),
    )(idx[:b_tc], tabT)

    tabF = jnp.pad(table.T, ((0, 0), (0, SCKP - V))).reshape(-1)
    offs = (jnp.arange(C, dtype=jnp.int32) * maxv)[None, :, None]
    idx_sc = idx[b_tc:] + offs

    info = plsc.get_sparse_core_info()
    nw = info.num_cores * info.num_subcores
    sc = functools.partial(
        pl.kernel,
        mesh=plsc.VectorSubcoreMesh(core_axis_name="c", subcore_axis_name="s"),
        out_type=jax.ShapeDtypeStruct((B_SC, D, P), jnp.float32),
        scratch_types=[
            pltpu.VMEM((D * SCKP,), jnp.float32),
            pltpu.VMEM((C, P), jnp.int32),
            pltpu.VMEM((DBLK, P), jnp.float32),
        ],
        compiler_params=pltpu.CompilerParams(needs_layout_passes=False),
    )
    body = functools.partial(
        _sc_body, b_per_w=B_SC // nw, n_chan=C, pixels=P, d_dim=D)
    out_sc = sc(body)(tabF, idx_sc)

    out = jnp.concatenate([out_tc, out_sc], axis=0)
    return out.reshape(B, D, H, W)


# speedup vs baseline: 6.4319x; 1.8223x over previous
import functools

import jax
import jax.numpy as jnp
from jax.experimental import pallas as pl
from jax.experimental.pallas import tpu as pltpu

MAXV = 147
KPAD = 160


def _body(idx_ref, tab_ref, out_ref, *, t_imgs, n_chan, kpad, pixels):
    iota = jax.lax.broadcasted_iota(jnp.int32, (kpad, pixels), 0)
    iota_bf = iota.astype(jnp.bfloat16)
    one = jnp.bfloat16(1.0)
    zero = jnp.bfloat16(0.0)
    for t in range(t_imgs):
        hots = []
        for c in range(n_chan):
            idx_bf = idx_ref[t, c, :].astype(jnp.bfloat16)
            d = iota_bf - idx_bf[None, :]
            hots.append(jnp.maximum(one - jnp.abs(d), zero))
        onehot = jnp.concatenate(hots, axis=0)
        out_ref[t] = jnp.dot(tab_ref[...], onehot,
                             preferred_element_type=jnp.float32)


@jax.jit
def kernel(inputs, table):
    B, C, H, W = inputs.shape
    V, D = table.shape
    P = H * W
    maxv = V // C

    idx = inputs.astype(jnp.int32).reshape(B, C, P)

    tab = table.reshape(C, maxv, D)
    tab = jnp.pad(tab, ((0, 0), (0, KPAD - maxv), (0, 0)))
    tabT = jnp.transpose(tab, (2, 0, 1)).reshape(D, C * KPAD)
    tabT = tabT.astype(jnp.bfloat16)

    T = 32
    grid = (B // T,)
    out = pl.pallas_call(
        functools.partial(_body, t_imgs=T, n_chan=C, kpad=KPAD, pixels=P),
        grid=grid,
        in_specs=[
            pl.BlockSpec((T, C, P), lambda i: (i, 0, 0)),
            pl.BlockSpec((D, C * KPAD), lambda i: (0, 0)),
        ],
        out_specs=pl.BlockSpec((T, D, P), lambda i: (i, 0, 0)),
        out_shape=jax.ShapeDtypeStruct((B, D, P), jnp.float32),
        compiler_params=pltpu.CompilerParams(
            dimension_semantics=("parallel",)),
    )(idx, tabT)
    return out.reshape(B, D, H, W)
